# baseline (speedup 1.0000x reference)
import functools

import jax
import jax.numpy as jnp
import numpy as np
from jax import lax
from jax.experimental import pallas as pl
from jax.experimental.pallas import tpu as pltpu
from jax.experimental.pallas import tpu_sc as plsc

N_EMBD = 128
HALF = N_EMBD // 2
L = 16
NC = 2
NS = 16
NW = NC * NS
IDX_CHUNK = 128


def _rotary_cs_table(seq_len):
    inv_freq = 1.0 / (10000.0 ** (np.arange(0, N_EMBD, 2, dtype=np.float32) / N_EMBD))
    freqs = np.arange(seq_len, dtype=np.float32)[:, None] * inv_freq[None, :]
    cs = np.concatenate([np.cos(freqs), np.sin(freqs)], axis=1)
    return jnp.asarray(cs.reshape(-1))


def _make_sc_kernel(batch, seq_len):
    total = batch * seq_len
    b_per_w = total // NW
    n_gather = b_per_w // IDX_CHUNK
    w_per_seq = seq_len // b_per_w

    mesh = plsc.VectorSubcoreMesh(
        core_axis_name="c", subcore_axis_name="s", num_cores=NC, num_subcores=NS
    )

    @functools.partial(
        pl.kernel,
        out_type=jax.ShapeDtypeStruct((total, N_EMBD), jnp.float32),
        mesh=mesh,
        scratch_types=[
            pltpu.VMEM((b_per_w,), jnp.int32),
            pltpu.VMEM((b_per_w, N_EMBD), jnp.float32),
            pltpu.VMEM((b_per_w * N_EMBD,), jnp.float32),
            pltpu.SemaphoreType.DMA,
            pltpu.SemaphoreType.DMA,
            pltpu.SemaphoreType.DMA,
            pltpu.SemaphoreType.DMA,
        ],
    )
    def sc_kernel(tok_hbm, w_hbm, cs_hbm, out_hbm,
                  idx_v, rows_v, cs_v, sem_t, sem_g0, sem_g1, sem_w):
        wid = lax.axis_index("s") * NC + lax.axis_index("c")
        base = wid * b_per_w
        bi = lax.div(wid, w_per_seq)
        pos_base = lax.rem(wid, w_per_seq) * b_per_w

        pltpu.sync_copy(tok_hbm.at[bi, pl.ds(pos_base, b_per_w)], idx_v)

        tab = pltpu.async_copy(
            cs_hbm.at[pl.ds(pos_base * N_EMBD, b_per_w * N_EMBD)], cs_v, sem_t
        )
        gsems = [sem_g0, sem_g1]
        gathers = [
            pltpu.async_copy(
                w_hbm.at[idx_v.at[pl.ds(g * IDX_CHUNK, IDX_CHUNK)]],
                rows_v.at[pl.ds(g * IDX_CHUNK, IDX_CHUNK)],
                gsems[g],
            )
            for g in range(n_gather)
        ]
        tab.wait()

        def rotate_one(t):
            cb = t * N_EMBD
            ts = [rows_v[t, pl.ds(j * L, L)] for j in range(N_EMBD // L)]
            cs = [cs_v[pl.ds(cb + j * L, L)] for j in range(N_EMBD // L)]
            half = HALF // L
            for j in range(half):
                rows_v[t, pl.ds(j * L, L)] = ts[j] * cs[j] - ts[j + half] * cs[j + half]
                rows_v[t, pl.ds((j + half) * L, L)] = (
                    ts[j + half] * cs[j] + ts[j] * cs[j + half]
                )

        UNROLL = 2

        writes = []
        for g in range(n_gather):
            gathers[g].wait()

            def rotate(i, carry, g=g):
                t = g * IDX_CHUNK + i * UNROLL
                for u in range(UNROLL):
                    rotate_one(t + u)
                return carry

            lax.fori_loop(0, IDX_CHUNK // UNROLL, rotate, 0)
            writes.append(
                pltpu.async_copy(
                    rows_v.at[pl.ds(g * IDX_CHUNK, IDX_CHUNK)],
                    out_hbm.at[pl.ds(base + g * IDX_CHUNK, IDX_CHUNK)],
                    sem_w,
                )
            )
        for w in writes:
            w.wait()

    return sc_kernel


def kernel(token, W):
    batch, seq_len = token.shape
    cs = _rotary_cs_table(seq_len)
    sc = _make_sc_kernel(batch, seq_len)
    out = sc(token, W, cs)
    return out.reshape(batch, seq_len, N_EMBD)


# speedup vs baseline: 1.0603x; 1.0603x over previous
import functools

import jax
import jax.numpy as jnp
import numpy as np
from jax import lax
from jax.experimental import pallas as pl
from jax.experimental.pallas import tpu as pltpu
from jax.experimental.pallas import tpu_sc as plsc

N_EMBD = 128
HALF = N_EMBD // 2
L = 16
NC = 2
NS = 16
NW = NC * NS


def _angle_tables(seq_len, p_per_w):
    inv_freq = 1.0 / (10000.0 ** (np.arange(0, N_EMBD, 2, dtype=np.float32) / N_EMBD))
    coarse = (np.arange(NW, dtype=np.float32) * p_per_w)[:, None] * inv_freq[None, :]
    fine = np.arange(p_per_w, dtype=np.float32)[:, None] * inv_freq[None, :]
    a = np.concatenate([np.cos(coarse), np.sin(coarse)], axis=1)
    b = np.concatenate([np.cos(fine), np.sin(fine)], axis=1)
    return jnp.asarray(a.reshape(-1)), jnp.asarray(b.reshape(-1))


def _make_sc_kernel(batch, seq_len):
    p_per_w = seq_len // NW
    nh = HALF // L

    mesh = plsc.VectorSubcoreMesh(
        core_axis_name="c", subcore_axis_name="s", num_cores=NC, num_subcores=NS
    )

    @functools.partial(
        pl.kernel,
        out_type=jax.ShapeDtypeStruct((batch, seq_len, N_EMBD), jnp.float32),
        mesh=mesh,
        scratch_types=[
            pltpu.VMEM((batch * p_per_w,), jnp.int32),
            pltpu.VMEM((batch, p_per_w, N_EMBD), jnp.float32),
            pltpu.VMEM((p_per_w, N_EMBD), jnp.float32),
            pltpu.VMEM((N_EMBD,), jnp.float32),
            pltpu.VMEM((p_per_w * N_EMBD,), jnp.float32),
            pltpu.SemaphoreType.DMA,
            pltpu.SemaphoreType.DMA,
            pltpu.SemaphoreType.DMA,
            pltpu.SemaphoreType.DMA,
        ],
    )
    def sc_kernel(tok_hbm, w_hbm, a_hbm, b_hbm, out_hbm,
                  idx_v, rows_v, cs_v, a_v, b_v, sem_i, sem_t, sem_g, sem_w):
        wid = lax.axis_index("s") * NC + lax.axis_index("c")
        pbase = wid * p_per_w

        idx_copies = [
            pltpu.async_copy(
                tok_hbm.at[b, pl.ds(pbase, p_per_w)],
                idx_v.at[pl.ds(b * p_per_w, p_per_w)],
                sem_i,
            )
            for b in range(batch)
        ]
        ta = pltpu.async_copy(a_hbm.at[pl.ds(wid * N_EMBD, N_EMBD)], a_v, sem_t)
        tb = pltpu.async_copy(b_hbm.at[:], b_v, sem_t)
        for c in idx_copies:
            c.wait()

        gathers = [
            pltpu.async_copy(
                w_hbm.at[idx_v.at[pl.ds(b * p_per_w, p_per_w)]],
                rows_v.at[b],
                sem_g,
            )
            for b in range(batch)
        ]
        ta.wait()
        tb.wait()

        ca = [a_v[pl.ds(j * L, L)] for j in range(nh)]
        sa = [a_v[pl.ds(HALF + j * L, L)] for j in range(nh)]

        def build(p, carry):
            pb = p * N_EMBD
            for j in range(nh):
                cb = b_v[pl.ds(pb + j * L, L)]
                sb = b_v[pl.ds(pb + HALF + j * L, L)]
                cs_v[p, pl.ds(j * L, L)] = ca[j] * cb - sa[j] * sb
                cs_v[p, pl.ds(HALF + j * L, L)] = sa[j] * cb + ca[j] * sb
            return carry

        lax.fori_loop(0, p_per_w, build, 0)

        writes = []
        for b in range(batch):
            gathers[b].wait()

            def rotate(p, carry, b=b):
                ts = [rows_v[b, p, pl.ds(j * L, L)] for j in range(N_EMBD // L)]
                cs = [cs_v[p, pl.ds(j * L, L)] for j in range(N_EMBD // L)]
                for j in range(nh):
                    rows_v[b, p, pl.ds(j * L, L)] = (
                        ts[j] * cs[j] - ts[j + nh] * cs[j + nh]
                    )
                    rows_v[b, p, pl.ds((j + nh) * L, L)] = (
                        ts[j + nh] * cs[j] + ts[j] * cs[j + nh]
                    )
                return carry

            lax.fori_loop(0, p_per_w, rotate, 0)
            writes.append(
                pltpu.async_copy(
                    rows_v.at[b], out_hbm.at[b, pl.ds(pbase, p_per_w)], sem_w
                )
            )
        for w in writes:
            w.wait()

    return sc_kernel


def kernel(token, W):
    batch, seq_len = token.shape
    a, b = _angle_tables(seq_len, seq_len // NW)
    sc = _make_sc_kernel(batch, seq_len)
    return sc(token, W, a, b)
